# baseline (speedup 1.0000x reference)
import functools

import jax
import jax.numpy as jnp
from jax import lax
from jax.experimental import pallas as pl
from jax.experimental.pallas import tpu as pltpu
from jax.experimental.pallas import tpu_sc as plsc

NC = 2
NS = 16
NW = NC * NS
EPB = 128
AGG_RING = 6


def _cdiv(a, b):
    return (a + b - 1) // b


def _make_deg_kernel(n_pad, nbt):
    rpt = n_pad // NS
    mesh = plsc.VectorSubcoreMesh(core_axis_name="c", subcore_axis_name="s")

    @functools.partial(
        pl.kernel,
        out_type=jax.ShapeDtypeStruct((NC, n_pad, 16), jnp.float32),
        mesh=mesh,
        compiler_params=pltpu.CompilerParams(use_tc_tiling_on_sc=False),
        scratch_types=[
            pltpu.VMEM((nbt, EPB), jnp.int32),
            pltpu.VMEM((EPB, 16), jnp.float32),
            pltpu.VMEM_SHARED((n_pad, 16), jnp.float32),
            pltpu.SemaphoreType.DMA,
            pltpu.SemaphoreType.DMA,
        ],
    )
    def deg_kernel(dst_hbm, z16_hbm, ones_hbm, out_hbm, idx_v, ones_v, acc,
                   isem, ssem):
        c = lax.axis_index("c")
        s = lax.axis_index("s")
        wid = c * NS + s
        pltpu.async_copy(dst_hbm.at[pl.ds(wid * nbt, nbt)], idx_v, isem)
        pltpu.sync_copy(ones_hbm, ones_v)
        pltpu.sync_copy(z16_hbm.at[pl.ds(s * rpt, rpt)],
                        acc.at[pl.ds(s * rpt, rpt)])
        pltpu.make_async_copy(dst_hbm.at[pl.ds(wid * nbt, nbt)], idx_v,
                              isem).wait()
        plsc.subcore_barrier()

        @pl.loop(0, nbt)
        def _(b):
            pltpu.sync_copy(ones_v, acc.at[idx_v.at[b]], add=True)

        plsc.subcore_barrier()
        pltpu.sync_copy(acc.at[pl.ds(s * rpt, rpt)],
                        out_hbm.at[c, pl.ds(s * rpt, rpt)])

    return deg_kernel


def _make_agg_kernel(n_pad, nbt, hf, ring=8):
    rpt = n_pad // NS
    mesh = plsc.VectorSubcoreMesh(core_axis_name="c", subcore_axis_name="s")

    half = ring // 2
    assert nbt % ring == 0 and ring % 2 == 0

    @functools.partial(
        pl.kernel,
        out_type=jax.ShapeDtypeStruct((NC, n_pad, hf), jnp.float32),
        mesh=mesh,
        compiler_params=pltpu.CompilerParams(use_tc_tiling_on_sc=False),
        scratch_types=(
            [pltpu.VMEM((nbt, EPB), jnp.int32)] * 2
            + [pltpu.VMEM((EPB, hf), jnp.float32)] * ring
            + [pltpu.SemaphoreType.DMA] * (2 * ring + 1)
            + [pltpu.VMEM_SHARED((n_pad, hf), jnp.float32)]
        ),
    )
    def agg_kernel(h_hbm, src_hbm, dst_hbm, z_hbm, out_hbm, *refs):
        src_v, dst_v = refs[0], refs[1]
        rows = refs[2:2 + ring]
        gsem = refs[2 + ring:2 + 2 * ring]
        ssem = refs[2 + 2 * ring:2 + 3 * ring]
        isem = refs[2 + 3 * ring]
        acc = refs[3 + 3 * ring]
        c = lax.axis_index("c")
        s = lax.axis_index("s")

        def gd(b, j):
            return pltpu.make_async_copy(h_hbm.at[src_v.at[b]], rows[j],
                                         gsem[j])

        def sd(b, j):
            return pltpu.make_async_copy(rows[j], acc.at[dst_v.at[b]],
                                         ssem[j])

        pltpu.async_copy(src_hbm.at[c, pl.ds(s * nbt, nbt)], src_v, isem)
        pltpu.async_copy(dst_hbm.at[pl.ds(s * nbt, nbt)], dst_v, isem)
        pltpu.sync_copy(z_hbm.at[pl.ds(s * rpt, rpt)],
                        acc.at[pl.ds(s * rpt, rpt)])
        pltpu.make_async_copy(src_hbm.at[c, pl.ds(s * nbt, nbt)], src_v,
                              isem).wait()
        pltpu.make_async_copy(dst_hbm.at[pl.ds(s * nbt, nbt)], dst_v,
                              isem).wait()
        plsc.subcore_barrier()

        for j in range(half):
            gd(j, j).start()

        @pl.loop(0, nbt // ring)
        def _(o):
            for j in range(ring):
                b = o * ring + j
                jj = (j + half) % ring
                bg = b + half

                @pl.when(bg >= ring)
                def _():
                    sd(bg - ring, jj).wait()

                @pl.when(bg < nbt)
                def _():
                    gd(bg, jj).start()

                gd(b, j).wait()
                sd(b, j).start(add=True)

        for b in range(nbt - half, nbt):
            sd(b, b % ring).wait()

        plsc.subcore_barrier()
        pltpu.sync_copy(acc.at[pl.ds(s * rpt, rpt)],
                        out_hbm.at[c, pl.ds(s * rpt, rpt)])

    return agg_kernel


def _prep1_body(x_ref, w_ref, d0_ref, d1_ref, hp_ref, dinv_ref):
    d = d0_ref[...] + d1_ref[...]
    cnt = d[:, 0:1] + 1.0
    dinv = lax.rsqrt(cnt)
    h = jnp.dot(x_ref[...], w_ref[...], preferred_element_type=jnp.float32,
                precision=lax.Precision.HIGHEST)
    hp_ref[...] = h * dinv
    dinv_ref[...] = jnp.broadcast_to(dinv, d.shape)


def _mid_body(alo_ref, ahi_ref, hp_ref, dinv_ref, b1_ref, w_ref, out_ref):
    dinv = dinv_ref[...][:, 0:1]
    agg = jnp.concatenate([alo_ref[...], ahi_ref[...]], axis=1)
    z = (agg + hp_ref[...]) * dinv + b1_ref[...]
    h = jnp.maximum(z, 0.0)
    h2 = jnp.dot(h, w_ref[...], preferred_element_type=jnp.float32,
                 precision=lax.Precision.HIGHEST)
    out_ref[...] = h2 * dinv


def _fin_body(alo_ref, ahi_ref, hp_ref, dinv_ref, b2_ref, out_ref):
    dinv = dinv_ref[...][:, 0:1]
    agg = jnp.concatenate([alo_ref[...], ahi_ref[...]], axis=1)
    out_ref[...] = (agg + hp_ref[...]) * dinv + b2_ref[...]


def kernel(x, edge_index, W1, b1, W2, b2):
    n, f_in = x.shape
    hid = W1.shape[1]
    hf = hid // 2
    e = edge_index.shape[1]

    ring = AGG_RING
    rnd = max(2, ring)
    nbt = _cdiv(_cdiv(e, EPB * NS), rnd) * rnd
    nbt_deg = nbt // 2
    e_pad = NS * nbt * EPB
    n_pad = _cdiv(n + 1, NS * 8) * NS * 8

    src = edge_index[0]
    dst = edge_index[1]
    pad = e_pad - e
    src_p = jnp.concatenate([src, jnp.zeros((pad,), jnp.int32)])
    dst_p = jnp.concatenate([dst, jnp.full((pad,), n, jnp.int32)])
    dst2d = dst_p.reshape(NS * nbt, EPB)
    src2 = jnp.stack(
        [(2 * src_p).reshape(NS * nbt, EPB),
         (2 * src_p + 1).reshape(NS * nbt, EPB)])

    zh = jnp.zeros((n_pad, hf), jnp.float32)
    z16 = jnp.zeros((n_pad, 16), jnp.float32)
    ones16 = jnp.ones((EPB, 16), jnp.float32)

    deg_kernel = _make_deg_kernel(n_pad, nbt_deg)
    agg_kernel = _make_agg_kernel(n_pad, nbt, hf, ring)

    degs = deg_kernel(dst2d, z16, ones16)

    rb = 1000
    grid = (n // rb,)
    blk = lambda shape, imap: pl.BlockSpec(shape, imap)
    row_map = lambda i: (i, 0)
    fix_map = lambda i: (0, 0)

    hp1, dinv16 = pl.pallas_call(
        _prep1_body,
        grid=grid,
        in_specs=[
            blk((rb, f_in), row_map),
            blk((f_in, hid), fix_map),
            blk((rb, 16), row_map),
            blk((rb, 16), row_map),
        ],
        out_specs=[blk((rb, hid), row_map), blk((rb, 16), row_map)],
        out_shape=[
            jax.ShapeDtypeStruct((n, hid), jnp.float32),
            jax.ShapeDtypeStruct((n, 16), jnp.float32),
        ],
    )(x, W1, degs[0], degs[1])

    acc1 = agg_kernel(hp1.reshape(2 * n, hf), src2, dst2d, zh)

    hp2 = pl.pallas_call(
        _mid_body,
        grid=grid,
        in_specs=[
            blk((rb, hf), row_map),
            blk((rb, hf), row_map),
            blk((rb, hid), row_map),
            blk((rb, 16), row_map),
            blk((1, hid), fix_map),
            blk((hid, hid), fix_map),
        ],
        out_specs=blk((rb, hid), row_map),
        out_shape=jax.ShapeDtypeStruct((n, hid), jnp.float32),
    )(acc1[0], acc1[1], hp1, dinv16, b1.reshape(1, hid), W2)

    acc2 = agg_kernel(hp2.reshape(2 * n, hf), src2, dst2d, zh)

    out = pl.pallas_call(
        _fin_body,
        grid=grid,
        in_specs=[
            blk((rb, hf), row_map),
            blk((rb, hf), row_map),
            blk((rb, hid), row_map),
            blk((rb, 16), row_map),
            blk((1, hid), fix_map),
        ],
        out_specs=blk((rb, hid), row_map),
        out_shape=jax.ShapeDtypeStruct((n, hid), jnp.float32),
    )(acc2[0], acc2[1], hp2, dinv16, b2.reshape(1, hid))

    return out


# speedup vs baseline: 1.3214x; 1.3214x over previous
import functools

import jax
import jax.numpy as jnp
from jax import lax
from jax.experimental import pallas as pl
from jax.experimental.pallas import tpu as pltpu
from jax.experimental.pallas import tpu_sc as plsc

NC = 2
NS = 16
NW = NC * NS
EPB = 128
RING = 2
AGG_GM = 2


def _cdiv(a, b):
    return (a + b - 1) // b


def _make_deg_kernel(n_pad, nbt):
    rpt = n_pad // NS
    mesh = plsc.VectorSubcoreMesh(core_axis_name="c", subcore_axis_name="s")

    @functools.partial(
        pl.kernel,
        out_type=jax.ShapeDtypeStruct((NC, n_pad, 16), jnp.float32),
        mesh=mesh,
        compiler_params=pltpu.CompilerParams(use_tc_tiling_on_sc=False),
        scratch_types=[
            pltpu.VMEM((nbt, EPB), jnp.int32),
            pltpu.VMEM((EPB, 16), jnp.float32),
            pltpu.VMEM_SHARED((n_pad, 16), jnp.float32),
            pltpu.SemaphoreType.DMA,
            pltpu.SemaphoreType.DMA,
        ],
    )
    def deg_kernel(dst_hbm, z16_hbm, ones_hbm, out_hbm, idx_v, ones_v, acc,
                   isem, ssem):
        c = lax.axis_index("c")
        s = lax.axis_index("s")
        wid = c * NS + s
        pltpu.async_copy(dst_hbm.at[pl.ds(wid * nbt, nbt)], idx_v, isem)
        pltpu.sync_copy(ones_hbm, ones_v)
        pltpu.sync_copy(z16_hbm.at[pl.ds(s * rpt, rpt)],
                        acc.at[pl.ds(s * rpt, rpt)])
        pltpu.make_async_copy(dst_hbm.at[pl.ds(wid * nbt, nbt)], idx_v,
                              isem).wait()
        plsc.subcore_barrier()

        @pl.loop(0, nbt)
        def _(b):
            pltpu.sync_copy(ones_v, acc.at[idx_v.at[b]], add=True)

        plsc.subcore_barrier()
        pltpu.sync_copy(acc.at[pl.ds(s * rpt, rpt)],
                        out_hbm.at[c, pl.ds(s * rpt, rpt)])

    return deg_kernel


def _make_agg_kernel(n_pad, nbt, hf, gm=1):
    rpt = n_pad // NS
    nbg = nbt // gm
    gw = gm * EPB
    mesh = plsc.VectorSubcoreMesh(core_axis_name="c", subcore_axis_name="s")

    @functools.partial(
        pl.kernel,
        out_type=jax.ShapeDtypeStruct((NC, n_pad, hf), jnp.float32),
        mesh=mesh,
        compiler_params=pltpu.CompilerParams(use_tc_tiling_on_sc=False),
        scratch_types=(
            [pltpu.VMEM((nbg, gw), jnp.int32),
             pltpu.VMEM((nbt, EPB), jnp.int32)]
            + [pltpu.VMEM((gw, hf), jnp.float32)] * RING
            + [pltpu.SemaphoreType.DMA] * (RING + 1)
            + [pltpu.VMEM_SHARED((n_pad, hf), jnp.float32)]
        ),
    )
    def agg_kernel(h_hbm, src_hbm, dst_hbm, z_hbm, out_hbm, *refs):
        src_v, dst_v = refs[0], refs[1]
        rows = refs[2:2 + RING]
        gsem = refs[2 + RING:2 + 2 * RING]
        isem = refs[2 + 2 * RING]
        acc = refs[3 + 2 * RING]
        c = lax.axis_index("c")
        s = lax.axis_index("s")
        pltpu.async_copy(src_hbm.at[c, pl.ds(s * nbg, nbg)], src_v, isem)
        pltpu.async_copy(dst_hbm.at[pl.ds(s * nbt, nbt)], dst_v, isem)
        pltpu.sync_copy(z_hbm.at[pl.ds(s * rpt, rpt)],
                        acc.at[pl.ds(s * rpt, rpt)])
        pltpu.make_async_copy(src_hbm.at[c, pl.ds(s * nbg, nbg)], src_v,
                              isem).wait()
        pltpu.make_async_copy(dst_hbm.at[pl.ds(s * nbt, nbt)], dst_v,
                              isem).wait()
        plsc.subcore_barrier()

        for j in range(RING):
            pltpu.make_async_copy(h_hbm.at[src_v.at[j]], rows[j],
                                  gsem[j]).start()

        n_outer = nbg // RING

        @pl.loop(0, n_outer)
        def _(o):
            for j in range(RING):
                g = o * RING + j
                pltpu.make_async_copy(h_hbm.at[src_v.at[g]], rows[j],
                                      gsem[j]).wait()
                for k in range(gm):
                    pltpu.sync_copy(rows[j].at[pl.ds(k * EPB, EPB)],
                                    acc.at[dst_v.at[g * gm + k]], add=True)

                @pl.when(o < n_outer - 1)
                def _():
                    pltpu.make_async_copy(h_hbm.at[src_v.at[g + RING]],
                                          rows[j], gsem[j]).start()

        plsc.subcore_barrier()
        pltpu.sync_copy(acc.at[pl.ds(s * rpt, rpt)],
                        out_hbm.at[c, pl.ds(s * rpt, rpt)])

    return agg_kernel


def _prep1_body(x_ref, w_ref, d0_ref, d1_ref, hp_ref, dinv_ref):
    d = d0_ref[...] + d1_ref[...]
    cnt = d[:, 0:1] + 1.0
    dinv = lax.rsqrt(cnt)
    h = jnp.dot(x_ref[...], w_ref[...], preferred_element_type=jnp.float32,
                precision=lax.Precision.HIGHEST)
    hp_ref[...] = h * dinv
    dinv_ref[...] = jnp.broadcast_to(dinv, d.shape)


def _mid_body(alo_ref, ahi_ref, hp_ref, dinv_ref, b1_ref, w_ref, out_ref):
    dinv = dinv_ref[...][:, 0:1]
    agg = jnp.concatenate([alo_ref[...], ahi_ref[...]], axis=1)
    z = (agg + hp_ref[...]) * dinv + b1_ref[...]
    h = jnp.maximum(z, 0.0)
    h2 = jnp.dot(h, w_ref[...], preferred_element_type=jnp.float32,
                 precision=lax.Precision.HIGHEST)
    out_ref[...] = h2 * dinv


def _fin_body(alo_ref, ahi_ref, hp_ref, dinv_ref, b2_ref, out_ref):
    dinv = dinv_ref[...][:, 0:1]
    agg = jnp.concatenate([alo_ref[...], ahi_ref[...]], axis=1)
    out_ref[...] = (agg + hp_ref[...]) * dinv + b2_ref[...]


def kernel(x, edge_index, W1, b1, W2, b2):
    n, f_in = x.shape
    hid = W1.shape[1]
    hf = hid // 2
    e = edge_index.shape[1]

    gm = AGG_GM
    rnd = max(2, gm * RING)
    nbt = _cdiv(_cdiv(e, EPB * NS), rnd) * rnd
    nbt_deg = nbt // 2
    e_pad = NS * nbt * EPB
    n_pad = _cdiv(n + 1, NS * 8) * NS * 8

    src = edge_index[0]
    dst = edge_index[1]
    pad = e_pad - e
    src_p = jnp.concatenate([src, jnp.zeros((pad,), jnp.int32)])
    dst2d = jnp.concatenate(
        [dst, jnp.full((pad,), n, jnp.int32)]).reshape(NS * nbt, EPB)
    nbg = nbt // gm
    src2 = jnp.stack(
        [(2 * src_p).reshape(NS * nbg, gm * EPB),
         (2 * src_p + 1).reshape(NS * nbg, gm * EPB)])

    zh = jnp.zeros((n_pad, hf), jnp.float32)
    z16 = jnp.zeros((n_pad, 16), jnp.float32)
    ones16 = jnp.ones((EPB, 16), jnp.float32)

    deg_kernel = _make_deg_kernel(n_pad, nbt_deg)
    agg_kernel = _make_agg_kernel(n_pad, nbt, hf, gm)

    degs = deg_kernel(dst2d, z16, ones16)

    rb = 1000
    grid = (n // rb,)
    blk = lambda shape, imap: pl.BlockSpec(shape, imap)
    row_map = lambda i: (i, 0)
    fix_map = lambda i: (0, 0)

    hp1, dinv16 = pl.pallas_call(
        _prep1_body,
        grid=grid,
        in_specs=[
            blk((rb, f_in), row_map),
            blk((f_in, hid), fix_map),
            blk((rb, 16), row_map),
            blk((rb, 16), row_map),
        ],
        out_specs=[blk((rb, hid), row_map), blk((rb, 16), row_map)],
        out_shape=[
            jax.ShapeDtypeStruct((n, hid), jnp.float32),
            jax.ShapeDtypeStruct((n, 16), jnp.float32),
        ],
    )(x, W1, degs[0], degs[1])

    acc1 = agg_kernel(hp1.reshape(2 * n, hf), src2, dst2d, zh)

    hp2 = pl.pallas_call(
        _mid_body,
        grid=grid,
        in_specs=[
            blk((rb, hf), row_map),
            blk((rb, hf), row_map),
            blk((rb, hid), row_map),
            blk((rb, 16), row_map),
            blk((1, hid), fix_map),
            blk((hid, hid), fix_map),
        ],
        out_specs=blk((rb, hid), row_map),
        out_shape=jax.ShapeDtypeStruct((n, hid), jnp.float32),
    )(acc1[0], acc1[1], hp1, dinv16, b1.reshape(1, hid), W2)

    acc2 = agg_kernel(hp2.reshape(2 * n, hf), src2, dst2d, zh)

    out = pl.pallas_call(
        _fin_body,
        grid=grid,
        in_specs=[
            blk((rb, hf), row_map),
            blk((rb, hf), row_map),
            blk((rb, hid), row_map),
            blk((rb, 16), row_map),
            blk((1, hid), fix_map),
        ],
        out_specs=blk((rb, hid), row_map),
        out_shape=jax.ShapeDtypeStruct((n, hid), jnp.float32),
    )(acc2[0], acc2[1], hp2, dinv16, b2.reshape(1, hid))

    return out


# speedup vs baseline: 1.3216x; 1.0001x over previous
import functools

import jax
import jax.numpy as jnp
from jax import lax
from jax.experimental import pallas as pl
from jax.experimental.pallas import tpu as pltpu
from jax.experimental.pallas import tpu_sc as plsc

NC = 2
NS = 16
NW = NC * NS
EPB = 128
RING = 2
AGG_GM = 2


def _cdiv(a, b):
    return (a + b - 1) // b


def _make_deg_kernel(n_pad, nbt):
    rpt = n_pad // NS
    mesh = plsc.VectorSubcoreMesh(core_axis_name="c", subcore_axis_name="s")

    @functools.partial(
        pl.kernel,
        out_type=jax.ShapeDtypeStruct((NC, n_pad, 16), jnp.float32),
        mesh=mesh,
        compiler_params=pltpu.CompilerParams(use_tc_tiling_on_sc=False),
        scratch_types=[
            pltpu.VMEM((nbt, EPB), jnp.int32),
            pltpu.VMEM((EPB, 16), jnp.float32),
            pltpu.VMEM_SHARED((n_pad, 16), jnp.float32),
            pltpu.SemaphoreType.DMA,
            pltpu.SemaphoreType.DMA,
        ],
    )
    def deg_kernel(dst_hbm, z16_hbm, ones_hbm, out_hbm, idx_v, ones_v, acc,
                   isem, ssem):
        c = lax.axis_index("c")
        s = lax.axis_index("s")
        wid = c * NS + s
        pltpu.async_copy(dst_hbm.at[pl.ds(wid * nbt, nbt)], idx_v, isem)
        pltpu.sync_copy(ones_hbm, ones_v)
        pltpu.sync_copy(z16_hbm.at[pl.ds(s * rpt, rpt)],
                        acc.at[pl.ds(s * rpt, rpt)])
        pltpu.make_async_copy(dst_hbm.at[pl.ds(wid * nbt, nbt)], idx_v,
                              isem).wait()
        plsc.subcore_barrier()

        @pl.loop(0, nbt)
        def _(b):
            pltpu.sync_copy(ones_v, acc.at[idx_v.at[b]], add=True)

        plsc.subcore_barrier()
        pltpu.sync_copy(acc.at[pl.ds(s * rpt, rpt)],
                        out_hbm.at[c, pl.ds(s * rpt, rpt)])

    return deg_kernel


def _make_agg_kernel(n_pad, nbt, hf, gm=1):
    rpt = n_pad // NS
    nbg = nbt // gm
    gw = gm * EPB
    mesh = plsc.VectorSubcoreMesh(core_axis_name="c", subcore_axis_name="s")

    @functools.partial(
        pl.kernel,
        out_type=jax.ShapeDtypeStruct((NC, n_pad, hf), jnp.float32),
        mesh=mesh,
        compiler_params=pltpu.CompilerParams(use_tc_tiling_on_sc=False),
        scratch_types=(
            [pltpu.VMEM((nbg, gw), jnp.int32),
             pltpu.VMEM((nbt, EPB), jnp.int32)]
            + [pltpu.VMEM((gw, hf), jnp.float32)] * RING
            + [pltpu.SemaphoreType.DMA] * (RING + 1)
            + [pltpu.VMEM_SHARED((n_pad, hf), jnp.float32)]
        ),
    )
    def agg_kernel(h_hbm, src_hbm, dst_hbm, z_hbm, out_hbm, *refs):
        src_v, dst_v = refs[0], refs[1]
        rows = refs[2:2 + RING]
        gsem = refs[2 + RING:2 + 2 * RING]
        isem = refs[2 + 2 * RING]
        acc = refs[3 + 2 * RING]
        c = lax.axis_index("c")
        s = lax.axis_index("s")
        pltpu.async_copy(src_hbm.at[c, pl.ds(s * nbg, nbg)], src_v, isem)
        pltpu.async_copy(dst_hbm.at[pl.ds(s * nbt, nbt)], dst_v, isem)
        pltpu.sync_copy(z_hbm.at[pl.ds(s * rpt, rpt)],
                        acc.at[pl.ds(s * rpt, rpt)])
        pltpu.make_async_copy(src_hbm.at[c, pl.ds(s * nbg, nbg)], src_v,
                              isem).wait()
        pltpu.make_async_copy(dst_hbm.at[pl.ds(s * nbt, nbt)], dst_v,
                              isem).wait()
        plsc.subcore_barrier()

        for j in range(RING):
            pltpu.make_async_copy(h_hbm.at[src_v.at[j]], rows[j],
                                  gsem[j]).start()

        n_outer = nbg // RING

        @pl.loop(0, n_outer)
        def _(o):
            for j in range(RING):
                g = o * RING + j
                pltpu.make_async_copy(h_hbm.at[src_v.at[g]], rows[j],
                                      gsem[j]).wait()
                for k in range(gm):
                    pltpu.sync_copy(rows[j].at[pl.ds(k * EPB, EPB)],
                                    acc.at[dst_v.at[g * gm + k]], add=True)

                @pl.when(o < n_outer - 1)
                def _():
                    pltpu.make_async_copy(h_hbm.at[src_v.at[g + RING]],
                                          rows[j], gsem[j]).start()

        plsc.subcore_barrier()
        pltpu.sync_copy(acc.at[pl.ds(s * rpt, rpt)],
                        out_hbm.at[c, pl.ds(s * rpt, rpt)])

    return agg_kernel


def _mm_body(x_ref, w_ref, h_ref):
    h_ref[...] = jnp.dot(x_ref[...], w_ref[...],
                         preferred_element_type=jnp.float32,
                         precision=lax.Precision.HIGHEST)


def _prep1_body(h_ref, d0_ref, d1_ref, hp_ref, dinv_ref):
    d = d0_ref[...] + d1_ref[...]
    cnt = d[:, 0:1] + 1.0
    dinv = lax.rsqrt(cnt)
    hp_ref[...] = h_ref[...] * dinv
    dinv_ref[...] = jnp.broadcast_to(dinv, d.shape)


def _mid_body(alo_ref, ahi_ref, hp_ref, dinv_ref, b1_ref, w_ref, out_ref):
    dinv = dinv_ref[...][:, 0:1]
    agg = jnp.concatenate([alo_ref[...], ahi_ref[...]], axis=1)
    z = (agg + hp_ref[...]) * dinv + b1_ref[...]
    h = jnp.maximum(z, 0.0)
    h2 = jnp.dot(h, w_ref[...], preferred_element_type=jnp.float32,
                 precision=lax.Precision.HIGHEST)
    out_ref[...] = h2 * dinv


def _fin_body(alo_ref, ahi_ref, hp_ref, dinv_ref, b2_ref, out_ref):
    dinv = dinv_ref[...][:, 0:1]
    agg = jnp.concatenate([alo_ref[...], ahi_ref[...]], axis=1)
    out_ref[...] = (agg + hp_ref[...]) * dinv + b2_ref[...]


def kernel(x, edge_index, W1, b1, W2, b2):
    n, f_in = x.shape
    hid = W1.shape[1]
    hf = hid // 2
    e = edge_index.shape[1]

    gm = AGG_GM
    rnd = max(2, gm * RING)
    nbt = _cdiv(_cdiv(e, EPB * NS), rnd) * rnd
    nbt_deg = nbt // 2
    e_pad = NS * nbt * EPB
    n_pad = _cdiv(n + 1, NS * 8) * NS * 8

    src = edge_index[0]
    dst = edge_index[1]
    pad = e_pad - e
    src_p = jnp.concatenate([src, jnp.zeros((pad,), jnp.int32)])
    dst2d = jnp.concatenate(
        [dst, jnp.full((pad,), n, jnp.int32)]).reshape(NS * nbt, EPB)
    nbg = nbt // gm
    src2 = jnp.stack(
        [(2 * src_p).reshape(NS * nbg, gm * EPB),
         (2 * src_p + 1).reshape(NS * nbg, gm * EPB)])

    zh = jnp.zeros((n_pad, hf), jnp.float32)
    z16 = jnp.zeros((n_pad, 16), jnp.float32)
    ones16 = jnp.ones((EPB, 16), jnp.float32)

    deg_kernel = _make_deg_kernel(n_pad, nbt_deg)
    agg_kernel = _make_agg_kernel(n_pad, nbt, hf, gm)

    degs = deg_kernel(dst2d, z16, ones16)

    rb = 1000
    grid = (n // rb,)
    blk = lambda shape, imap: pl.BlockSpec(shape, imap)
    row_map = lambda i: (i, 0)
    fix_map = lambda i: (0, 0)

    h1raw = pl.pallas_call(
        _mm_body,
        grid=grid,
        in_specs=[blk((rb, f_in), row_map), blk((f_in, hid), fix_map)],
        out_specs=blk((rb, hid), row_map),
        out_shape=jax.ShapeDtypeStruct((n, hid), jnp.float32),
    )(x, W1)

    hp1, dinv16 = pl.pallas_call(
        _prep1_body,
        grid=grid,
        in_specs=[
            blk((rb, hid), row_map),
            blk((rb, 16), row_map),
            blk((rb, 16), row_map),
        ],
        out_specs=[blk((rb, hid), row_map), blk((rb, 16), row_map)],
        out_shape=[
            jax.ShapeDtypeStruct((n, hid), jnp.float32),
            jax.ShapeDtypeStruct((n, 16), jnp.float32),
        ],
    )(h1raw, degs[0], degs[1])

    acc1 = agg_kernel(hp1.reshape(2 * n, hf), src2, dst2d, zh)

    hp2 = pl.pallas_call(
        _mid_body,
        grid=grid,
        in_specs=[
            blk((rb, hf), row_map),
            blk((rb, hf), row_map),
            blk((rb, hid), row_map),
            blk((rb, 16), row_map),
            blk((1, hid), fix_map),
            blk((hid, hid), fix_map),
        ],
        out_specs=blk((rb, hid), row_map),
        out_shape=jax.ShapeDtypeStruct((n, hid), jnp.float32),
    )(acc1[0], acc1[1], hp1, dinv16, b1.reshape(1, hid), W2)

    acc2 = agg_kernel(hp2.reshape(2 * n, hf), src2, dst2d, zh)

    out = pl.pallas_call(
        _fin_body,
        grid=grid,
        in_specs=[
            blk((rb, hf), row_map),
            blk((rb, hf), row_map),
            blk((rb, hid), row_map),
            blk((rb, 16), row_map),
            blk((1, hid), fix_map),
        ],
        out_specs=blk((rb, hid), row_map),
        out_shape=jax.ShapeDtypeStruct((n, hid), jnp.float32),
    )(acc2[0], acc2[1], hp2, dinv16, b2.reshape(1, hid))

    return out


# speedup vs baseline: 1.3445x; 1.0174x over previous
import functools

import jax
import jax.numpy as jnp
from jax import lax
from jax.experimental import pallas as pl
from jax.experimental.pallas import tpu as pltpu
from jax.experimental.pallas import tpu_sc as plsc

NC = 2
NS = 16
NW = NC * NS
EPB = 128
RING = 4
AGG_GM = 1


def _cdiv(a, b):
    return (a + b - 1) // b


def _make_deg_kernel(n_pad, nbt):
    rpt = n_pad // NS
    mesh = plsc.VectorSubcoreMesh(core_axis_name="c", subcore_axis_name="s")

    @functools.partial(
        pl.kernel,
        out_type=jax.ShapeDtypeStruct((NC, n_pad, 16), jnp.float32),
        mesh=mesh,
        compiler_params=pltpu.CompilerParams(use_tc_tiling_on_sc=False),
        scratch_types=[
            pltpu.VMEM((nbt, EPB), jnp.int32),
            pltpu.VMEM((EPB, 16), jnp.float32),
            pltpu.VMEM_SHARED((n_pad, 16), jnp.float32),
            pltpu.SemaphoreType.DMA,
            pltpu.SemaphoreType.DMA,
        ],
    )
    def deg_kernel(dst_hbm, z16_hbm, ones_hbm, out_hbm, idx_v, ones_v, acc,
                   isem, ssem):
        c = lax.axis_index("c")
        s = lax.axis_index("s")
        wid = c * NS + s
        pltpu.async_copy(dst_hbm.at[pl.ds(wid * nbt, nbt)], idx_v, isem)
        pltpu.sync_copy(ones_hbm, ones_v)
        pltpu.sync_copy(z16_hbm.at[pl.ds(s * rpt, rpt)],
                        acc.at[pl.ds(s * rpt, rpt)])
        pltpu.make_async_copy(dst_hbm.at[pl.ds(wid * nbt, nbt)], idx_v,
                              isem).wait()
        plsc.subcore_barrier()

        @pl.loop(0, nbt)
        def _(b):
            pltpu.sync_copy(ones_v, acc.at[idx_v.at[b]], add=True)

        plsc.subcore_barrier()
        pltpu.sync_copy(acc.at[pl.ds(s * rpt, rpt)],
                        out_hbm.at[c, pl.ds(s * rpt, rpt)])

    return deg_kernel


def _make_agg_kernel(n_pad, nbt, hf, gm=1):
    rpt = n_pad // NS
    nbg = nbt // gm
    gw = gm * EPB
    mesh = plsc.VectorSubcoreMesh(core_axis_name="c", subcore_axis_name="s")

    @functools.partial(
        pl.kernel,
        out_type=jax.ShapeDtypeStruct((NC, n_pad, hf), jnp.float32),
        mesh=mesh,
        compiler_params=pltpu.CompilerParams(use_tc_tiling_on_sc=False),
        scratch_types=(
            [pltpu.VMEM((nbg, gw), jnp.int32),
             pltpu.VMEM((nbt, EPB), jnp.int32)]
            + [pltpu.VMEM((gw, hf), jnp.float32)] * RING
            + [pltpu.SemaphoreType.DMA] * (RING + 1)
            + [pltpu.VMEM_SHARED((n_pad, hf), jnp.float32)]
        ),
    )
    def agg_kernel(h_hbm, src_hbm, dst_hbm, z_hbm, out_hbm, *refs):
        src_v, dst_v = refs[0], refs[1]
        rows = refs[2:2 + RING]
        gsem = refs[2 + RING:2 + 2 * RING]
        isem = refs[2 + 2 * RING]
        acc = refs[3 + 2 * RING]
        c = lax.axis_index("c")
        s = lax.axis_index("s")
        pltpu.async_copy(src_hbm.at[c, pl.ds(s * nbg, nbg)], src_v, isem)
        pltpu.async_copy(dst_hbm.at[pl.ds(s * nbt, nbt)], dst_v, isem)
        pltpu.sync_copy(z_hbm.at[pl.ds(s * rpt, rpt)],
                        acc.at[pl.ds(s * rpt, rpt)])
        pltpu.make_async_copy(src_hbm.at[c, pl.ds(s * nbg, nbg)], src_v,
                              isem).wait()
        pltpu.make_async_copy(dst_hbm.at[pl.ds(s * nbt, nbt)], dst_v,
                              isem).wait()
        plsc.subcore_barrier()

        for j in range(RING):
            pltpu.make_async_copy(h_hbm.at[src_v.at[j]], rows[j],
                                  gsem[j]).start()

        n_outer = nbg // RING

        @pl.loop(0, n_outer)
        def _(o):
            for j in range(RING):
                g = o * RING + j
                pltpu.make_async_copy(h_hbm.at[src_v.at[g]], rows[j],
                                      gsem[j]).wait()
                for k in range(gm):
                    pltpu.sync_copy(rows[j].at[pl.ds(k * EPB, EPB)],
                                    acc.at[dst_v.at[g * gm + k]], add=True)

                @pl.when(o < n_outer - 1)
                def _():
                    pltpu.make_async_copy(h_hbm.at[src_v.at[g + RING]],
                                          rows[j], gsem[j]).start()

        plsc.subcore_barrier()
        pltpu.sync_copy(acc.at[pl.ds(s * rpt, rpt)],
                        out_hbm.at[c, pl.ds(s * rpt, rpt)])

    return agg_kernel


def _mm_body(x_ref, w_ref, h_ref):
    h_ref[...] = jnp.dot(x_ref[...], w_ref[...],
                         preferred_element_type=jnp.float32,
                         precision=lax.Precision.HIGHEST)


def _prep1_body(h_ref, d0_ref, d1_ref, hp_ref, dinv_ref):
    d = d0_ref[...] + d1_ref[...]
    cnt = d[:, 0:1] + 1.0
    dinv = lax.rsqrt(cnt)
    hp_ref[...] = h_ref[...] * dinv
    dinv_ref[...] = jnp.broadcast_to(dinv, d.shape)


def _mid_body(alo_ref, ahi_ref, hp_ref, dinv_ref, b1_ref, w_ref, out_ref):
    dinv = dinv_ref[...][:, 0:1]
    agg = jnp.concatenate([alo_ref[...], ahi_ref[...]], axis=1)
    z = (agg + hp_ref[...]) * dinv + b1_ref[...]
    h = jnp.maximum(z, 0.0)
    h2 = jnp.dot(h, w_ref[...], preferred_element_type=jnp.float32,
                 precision=lax.Precision.HIGHEST)
    out_ref[...] = h2 * dinv


def _fin_body(alo_ref, ahi_ref, hp_ref, dinv_ref, b2_ref, out_ref):
    dinv = dinv_ref[...][:, 0:1]
    agg = jnp.concatenate([alo_ref[...], ahi_ref[...]], axis=1)
    out_ref[...] = (agg + hp_ref[...]) * dinv + b2_ref[...]


def kernel(x, edge_index, W1, b1, W2, b2):
    n, f_in = x.shape
    hid = W1.shape[1]
    hf = hid // 2
    e = edge_index.shape[1]

    gm = AGG_GM
    rnd = max(2, gm * RING)
    nbt = _cdiv(_cdiv(e, EPB * NS), rnd) * rnd
    nbt_deg = nbt // 2
    e_pad = NS * nbt * EPB
    n_pad = _cdiv(n + 1, NS * 8) * NS * 8

    src = edge_index[0]
    dst = edge_index[1]
    pad = e_pad - e
    src_p = jnp.concatenate([src, jnp.zeros((pad,), jnp.int32)])
    dst2d = jnp.concatenate(
        [dst, jnp.full((pad,), n, jnp.int32)]).reshape(NS * nbt, EPB)
    nbg = nbt // gm
    src2 = jnp.stack(
        [(2 * src_p).reshape(NS * nbg, gm * EPB),
         (2 * src_p + 1).reshape(NS * nbg, gm * EPB)])

    zh = jnp.zeros((n_pad, hf), jnp.float32)
    z16 = jnp.zeros((n_pad, 16), jnp.float32)
    ones16 = jnp.ones((EPB, 16), jnp.float32)

    deg_kernel = _make_deg_kernel(n_pad, nbt_deg)
    agg_kernel = _make_agg_kernel(n_pad, nbt, hf, gm)

    degs = deg_kernel(dst2d, z16, ones16)

    rb = 1000
    grid = (n // rb,)
    blk = lambda shape, imap: pl.BlockSpec(shape, imap)
    row_map = lambda i: (i, 0)
    fix_map = lambda i: (0, 0)

    h1raw = pl.pallas_call(
        _mm_body,
        grid=grid,
        in_specs=[blk((rb, f_in), row_map), blk((f_in, hid), fix_map)],
        out_specs=blk((rb, hid), row_map),
        out_shape=jax.ShapeDtypeStruct((n, hid), jnp.float32),
    )(x, W1)

    hp1, dinv16 = pl.pallas_call(
        _prep1_body,
        grid=grid,
        in_specs=[
            blk((rb, hid), row_map),
            blk((rb, 16), row_map),
            blk((rb, 16), row_map),
        ],
        out_specs=[blk((rb, hid), row_map), blk((rb, 16), row_map)],
        out_shape=[
            jax.ShapeDtypeStruct((n, hid), jnp.float32),
            jax.ShapeDtypeStruct((n, 16), jnp.float32),
        ],
    )(h1raw, degs[0], degs[1])

    acc1 = agg_kernel(hp1.reshape(2 * n, hf), src2, dst2d, zh)

    hp2 = pl.pallas_call(
        _mid_body,
        grid=grid,
        in_specs=[
            blk((rb, hf), row_map),
            blk((rb, hf), row_map),
            blk((rb, hid), row_map),
            blk((rb, 16), row_map),
            blk((1, hid), fix_map),
            blk((hid, hid), fix_map),
        ],
        out_specs=blk((rb, hid), row_map),
        out_shape=jax.ShapeDtypeStruct((n, hid), jnp.float32),
    )(acc1[0], acc1[1], hp1, dinv16, b1.reshape(1, hid), W2)

    acc2 = agg_kernel(hp2.reshape(2 * n, hf), src2, dst2d, zh)

    out = pl.pallas_call(
        _fin_body,
        grid=grid,
        in_specs=[
            blk((rb, hf), row_map),
            blk((rb, hf), row_map),
            blk((rb, hid), row_map),
            blk((rb, 16), row_map),
            blk((1, hid), fix_map),
        ],
        out_specs=blk((rb, hid), row_map),
        out_shape=jax.ShapeDtypeStruct((n, hid), jnp.float32),
    )(acc2[0], acc2[1], hp2, dinv16, b2.reshape(1, hid))

    return out
